# Initial kernel scaffold; baseline (speedup 1.0000x reference)
import jax
import jax.numpy as jnp
from jax.experimental import pallas as pl


def kernel(node_feats, edge_feats, edge_index, W_in, b_in, W_node, W_edge, W_new, b_new):
    raise NotImplementedError("write your pallas kernel here")


# speedup vs baseline: 2.3718x; 2.3718x over previous
import functools

import jax
import jax.numpy as jnp
from jax import lax
from jax.experimental import pallas as pl
from jax.experimental.pallas import tpu as pltpu
from jax.experimental.pallas import tpu_sc as plsc

N = 10000
E = 320000
NODE_IN = 55
D = 128
L = 4

NC = 2
NS = 16
NW = NC * NS

CH = 80
EDGES_PER_CORE = E // NC
EDGES_PER_TILE = EDGES_PER_CORE // NS
CHUNKS = EDGES_PER_TILE // CH
N_PAD = 10240
ROWS_PER_TILE = N_PAD // NS
ZROWS = 320

BN = 2000
BE = 10000


def _proj_body(nf_ref, win_ref, bin_ref, wn0_ref, h_ref, hw_ref):
    h = jnp.maximum(
        jnp.dot(nf_ref[...], win_ref[...], preferred_element_type=jnp.float32)
        + bin_ref[...], 0.0)
    h_ref[...] = h
    hw_ref[...] = jnp.dot(h, wn0_ref[...], preferred_element_type=jnp.float32)


def _project(node_feats, W_in, b_in, W_node0):
    return pl.pallas_call(
        _proj_body,
        grid=(N // BN,),
        in_specs=[
            pl.BlockSpec((BN, NODE_IN), lambda i: (i, 0)),
            pl.BlockSpec((NODE_IN, D), lambda i: (0, 0)),
            pl.BlockSpec((1, D), lambda i: (0, 0)),
            pl.BlockSpec((D, D), lambda i: (0, 0)),
        ],
        out_specs=[
            pl.BlockSpec((BN, D), lambda i: (i, 0)),
            pl.BlockSpec((BN, D), lambda i: (i, 0)),
        ],
        out_shape=[
            jax.ShapeDtypeStruct((N, D), jnp.float32),
            jax.ShapeDtypeStruct((N, D), jnp.float32),
        ],
    )(node_feats, W_in, b_in.reshape(1, D), W_node0)


def _edge_body(ef_ref, we_ref, out_ref):
    out_ref[0] = jnp.dot(ef_ref[...], we_ref[0],
                         preferred_element_type=jnp.float32)


def _edge_transform(edge_feats, W_edge):
    ein = edge_feats.shape[1]
    return pl.pallas_call(
        _edge_body,
        grid=(L, E // BE),
        in_specs=[
            pl.BlockSpec((BE, ein), lambda l, e: (e, 0)),
            pl.BlockSpec((1, ein, D), lambda l, e: (l, 0, 0)),
        ],
        out_specs=pl.BlockSpec((1, BE, D), lambda l, e: (l, e, 0)),
        out_shape=jax.ShapeDtypeStruct((L, E, D), jnp.float32),
    )(edge_feats, W_edge)


def _upd_body(h_ref, p_ref, wt_ref, wb_ref, b_ref, wn_ref, hnew_ref, hw_ref):
    p = p_ref[0] + p_ref[1]
    hn = jnp.maximum(
        jnp.dot(h_ref[...], wt_ref[...], preferred_element_type=jnp.float32)
        + jnp.dot(p, wb_ref[...], preferred_element_type=jnp.float32)
        + b_ref[...], 0.0)
    hnew_ref[...] = hn
    hw_ref[...] = jnp.dot(hn, wn_ref[...], preferred_element_type=jnp.float32)


def _update(h, parts, W_top, W_bot, b, W_node_next):
    return pl.pallas_call(
        _upd_body,
        grid=(N // BN,),
        in_specs=[
            pl.BlockSpec((BN, D), lambda i: (i, 0)),
            pl.BlockSpec((NC, BN, D), lambda i: (0, i, 0)),
            pl.BlockSpec((D, D), lambda i: (0, 0)),
            pl.BlockSpec((D, D), lambda i: (0, 0)),
            pl.BlockSpec((1, D), lambda i: (0, 0)),
            pl.BlockSpec((D, D), lambda i: (0, 0)),
        ],
        out_specs=[
            pl.BlockSpec((BN, D), lambda i: (i, 0)),
            pl.BlockSpec((BN, D), lambda i: (i, 0)),
        ],
        out_shape=[
            jax.ShapeDtypeStruct((N, D), jnp.float32),
            jax.ShapeDtypeStruct((N, D), jnp.float32),
        ],
    )(h, parts, W_top, W_bot, b.reshape(1, D), W_node_next)


def _sc_body(hw_hbm, ew_hbm, src_hbm, dst_hbm, out_hbm,
             acc, src_v, dst_v, rows_v, ew_v, sem):
    c = lax.axis_index("c")
    s = lax.axis_index("s")

    zeros16 = jnp.zeros((16,), jnp.float32)

    def zfill(i, _):
        for j in range(D // 16):
            rows_v[i, pl.ds(j * 16, 16)] = zeros16
        return 0

    lax.fori_loop(0, CH, zfill, 0)
    for j in range(ROWS_PER_TILE // CH):
        pltpu.sync_copy(rows_v, acc.at[pl.ds(s * ROWS_PER_TILE + j * CH, CH)])
    plsc.subcore_barrier()

    base0 = c * EDGES_PER_CORE + s * EDGES_PER_TILE

    def chunk(i, _):
        base = base0 + i * CH
        pltpu.sync_copy(src_hbm.at[pl.ds(base, CH)], src_v)
        pltpu.sync_copy(dst_hbm.at[pl.ds(base, CH)], dst_v)
        gcp = pltpu.async_copy(hw_hbm.at[src_v], rows_v, sem)
        pltpu.sync_copy(ew_hbm.at[pl.ds(base, CH)], ew_v)
        gcp.wait()

        def mul(e, _):
            for j in range(D // 16):
                sl = pl.ds(j * 16, 16)
                rows_v[e, sl] = rows_v[e, sl] * ew_v[e, sl]
            return 0

        lax.fori_loop(0, CH, mul, 0)
        pltpu.sync_copy(rows_v, acc.at[dst_v], add=True)
        return 0

    lax.fori_loop(0, CHUNKS, chunk, 0)
    plsc.subcore_barrier()

    pltpu.sync_copy(acc.at[pl.ds(s * ROWS_PER_TILE, ROWS_PER_TILE)],
                    out_hbm.at[c, pl.ds(s * ROWS_PER_TILE, ROWS_PER_TILE)])


def _sc_message_pass(hw, ew, src, dst):
    mesh = plsc.VectorSubcoreMesh(core_axis_name="c", subcore_axis_name="s")
    out = pl.kernel(
        _sc_body,
        out_type=jax.ShapeDtypeStruct((NC, N_PAD, D), jnp.float32),
        mesh=mesh,
        scratch_types=[
            pltpu.VMEM_SHARED((N_PAD, D), jnp.float32),
            pltpu.VMEM((CH,), jnp.int32),
            pltpu.VMEM((CH,), jnp.int32),
            pltpu.VMEM((CH, D), jnp.float32),
            pltpu.VMEM((CH, D), jnp.float32),
            pltpu.SemaphoreType.DMA,
        ],
    )(hw, ew, src, dst)
    return out[:, :N, :]


def kernel(node_feats, edge_feats, edge_index, W_in, b_in, W_node, W_edge,
           W_new, b_new):
    src = edge_index[0]
    dst = edge_index[1]
    ew_all = _edge_transform(edge_feats, W_edge)
    h, hw = _project(node_feats, W_in, b_in, W_node[0])
    for l in range(L):
        parts = _sc_message_pass(hw, ew_all[l], src, dst)
        wn_next = W_node[(l + 1) % L]
        h, hw = _update(h, parts, W_new[l][:D], W_new[l][D:], b_new[l], wn_next)
    return h


# speedup vs baseline: 2.8097x; 1.1846x over previous
import functools

import jax
import jax.numpy as jnp
from jax import lax
from jax.experimental import pallas as pl
from jax.experimental.pallas import tpu as pltpu
from jax.experimental.pallas import tpu_sc as plsc

N = 10000
E = 320000
NODE_IN = 55
D = 128
L = 4

NC = 2
NS = 16
NW = NC * NS

CH = 80
EDGES_PER_CORE = E // NC
EDGES_PER_TILE = EDGES_PER_CORE // NS
CHUNKS = EDGES_PER_TILE // CH
N_PAD = 10240
ROWS_PER_TILE = N_PAD // NS
ZROWS = 320

BN = 2000
BE = 10000


def _proj_body(nf_ref, win_ref, bin_ref, wn0_ref, h_ref, hw_ref):
    h = jnp.maximum(
        jnp.dot(nf_ref[...], win_ref[...], preferred_element_type=jnp.float32)
        + bin_ref[...], 0.0)
    h_ref[...] = h
    hw_ref[...] = jnp.dot(h, wn0_ref[...], preferred_element_type=jnp.float32)


def _project(node_feats, W_in, b_in, W_node0):
    return pl.pallas_call(
        _proj_body,
        grid=(N // BN,),
        in_specs=[
            pl.BlockSpec((BN, NODE_IN), lambda i: (i, 0)),
            pl.BlockSpec((NODE_IN, D), lambda i: (0, 0)),
            pl.BlockSpec((1, D), lambda i: (0, 0)),
            pl.BlockSpec((D, D), lambda i: (0, 0)),
        ],
        out_specs=[
            pl.BlockSpec((BN, D), lambda i: (i, 0)),
            pl.BlockSpec((BN, D), lambda i: (i, 0)),
        ],
        out_shape=[
            jax.ShapeDtypeStruct((N, D), jnp.float32),
            jax.ShapeDtypeStruct((N, D), jnp.float32),
        ],
    )(node_feats, W_in, b_in.reshape(1, D), W_node0)


def _edge_body(ef_ref, we_ref, out_ref):
    out_ref[0] = jnp.dot(ef_ref[...], we_ref[0],
                         preferred_element_type=jnp.float32)


def _edge_transform(edge_feats, W_edge):
    ein = edge_feats.shape[1]
    return pl.pallas_call(
        _edge_body,
        grid=(L, E // BE),
        in_specs=[
            pl.BlockSpec((BE, ein), lambda l, e: (e, 0)),
            pl.BlockSpec((1, ein, D), lambda l, e: (l, 0, 0)),
        ],
        out_specs=pl.BlockSpec((1, BE, D), lambda l, e: (l, e, 0)),
        out_shape=jax.ShapeDtypeStruct((L, E, D), jnp.float32),
    )(edge_feats, W_edge)


def _upd_body(h_ref, p_ref, wt_ref, wb_ref, b_ref, wn_ref, hnew_ref, hw_ref):
    p = p_ref[0] + p_ref[1]
    hn = jnp.maximum(
        jnp.dot(h_ref[...], wt_ref[...], preferred_element_type=jnp.float32)
        + jnp.dot(p, wb_ref[...], preferred_element_type=jnp.float32)
        + b_ref[...], 0.0)
    hnew_ref[...] = hn
    hw_ref[...] = jnp.dot(hn, wn_ref[...], preferred_element_type=jnp.float32)


def _update(h, parts, W_top, W_bot, b, W_node_next):
    return pl.pallas_call(
        _upd_body,
        grid=(N // BN,),
        in_specs=[
            pl.BlockSpec((BN, D), lambda i: (i, 0)),
            pl.BlockSpec((NC, BN, D), lambda i: (0, i, 0)),
            pl.BlockSpec((D, D), lambda i: (0, 0)),
            pl.BlockSpec((D, D), lambda i: (0, 0)),
            pl.BlockSpec((1, D), lambda i: (0, 0)),
            pl.BlockSpec((D, D), lambda i: (0, 0)),
        ],
        out_specs=[
            pl.BlockSpec((BN, D), lambda i: (i, 0)),
            pl.BlockSpec((BN, D), lambda i: (i, 0)),
        ],
        out_shape=[
            jax.ShapeDtypeStruct((N, D), jnp.float32),
            jax.ShapeDtypeStruct((N, D), jnp.float32),
        ],
    )(h, parts, W_top, W_bot, b.reshape(1, D), W_node_next)


def _sc_body(layer, hw_hbm, ew_hbm, src_hbm, dst_hbm, out_hbm,
             acc, src_v, dst_v, rows_v, ew_v, sem):
    c = lax.axis_index("c")
    s = lax.axis_index("s")

    zeros16 = jnp.zeros((16,), jnp.float32)

    def zfill(i, _):
        for j in range(D // 16):
            rows_v[i, pl.ds(j * 16, 16)] = zeros16
        return 0

    lax.fori_loop(0, CH, zfill, 0)
    for j in range(ROWS_PER_TILE // CH):
        pltpu.sync_copy(rows_v, acc.at[pl.ds(s * ROWS_PER_TILE + j * CH, CH)])
    plsc.subcore_barrier()

    base0 = c * EDGES_PER_CORE + s * EDGES_PER_TILE

    def chunk(i, _):
        base = base0 + i * CH
        pltpu.sync_copy(src_hbm.at[pl.ds(base, CH)], src_v)
        pltpu.sync_copy(dst_hbm.at[pl.ds(base, CH)], dst_v)
        gcp = pltpu.async_copy(hw_hbm.at[src_v], rows_v, sem)
        pltpu.sync_copy(ew_hbm.at[pl.ds(layer * E + base, CH)], ew_v)
        gcp.wait()

        def mul(e, _):
            for j in range(D // 16):
                sl = pl.ds(j * 16, 16)
                rows_v[e, sl] = rows_v[e, sl] * ew_v[e, sl]
            return 0

        lax.fori_loop(0, CH, mul, 0)
        pltpu.sync_copy(rows_v, acc.at[dst_v], add=True)
        return 0

    lax.fori_loop(0, CHUNKS, chunk, 0)
    plsc.subcore_barrier()

    pltpu.sync_copy(acc.at[pl.ds(s * ROWS_PER_TILE, ROWS_PER_TILE)],
                    out_hbm.at[c, pl.ds(s * ROWS_PER_TILE, ROWS_PER_TILE)])


def _sc_message_pass(layer, hw, ew_flat, src, dst):
    mesh = plsc.VectorSubcoreMesh(core_axis_name="c", subcore_axis_name="s")
    return pl.kernel(
        functools.partial(_sc_body, layer),
        out_type=jax.ShapeDtypeStruct((NC, N_PAD, D), jnp.float32),
        mesh=mesh,
        scratch_types=[
            pltpu.VMEM_SHARED((N_PAD, D), jnp.float32),
            pltpu.VMEM((CH,), jnp.int32),
            pltpu.VMEM((CH,), jnp.int32),
            pltpu.VMEM((CH, D), jnp.float32),
            pltpu.VMEM((CH, D), jnp.float32),
            pltpu.SemaphoreType.DMA,
        ],
    )(hw, ew_flat, src, dst)


def kernel(node_feats, edge_feats, edge_index, W_in, b_in, W_node, W_edge,
           W_new, b_new):
    src = edge_index[0]
    dst = edge_index[1]
    ew_flat = _edge_transform(edge_feats, W_edge).reshape(L * E, D)
    h, hw = _project(node_feats, W_in, b_in, W_node[0])
    for l in range(L):
        parts = _sc_message_pass(l, hw, ew_flat, src, dst)
        wn_next = W_node[(l + 1) % L]
        h, hw = _update(h, parts, W_new[l][:D], W_new[l][D:], b_new[l], wn_next)
    return h


# speedup vs baseline: 3.9073x; 1.3906x over previous
import functools

import jax
import jax.numpy as jnp
from jax import lax
from jax.experimental import pallas as pl
from jax.experimental.pallas import tpu as pltpu
from jax.experimental.pallas import tpu_sc as plsc

N = 10000
E = 320000
NODE_IN = 55
D = 128
L = 4

NC = 2
NS = 16
NW = NC * NS

CH = 80
EDGES_PER_CORE = E // NC
EDGES_PER_TILE = EDGES_PER_CORE // NS
CHUNKS = EDGES_PER_TILE // CH
N_PAD = 10240
ROWS_PER_TILE = N_PAD // NS
ZROWS = 320

BN = 2000
BE = 10000


def _proj_body(nf_ref, win_ref, bin_ref, wn0_ref, h_ref, hw_ref):
    h = jnp.maximum(
        jnp.dot(nf_ref[...], win_ref[...], preferred_element_type=jnp.float32)
        + bin_ref[...], 0.0)
    h_ref[...] = h
    hw_ref[...] = jnp.dot(h, wn0_ref[...], preferred_element_type=jnp.float32)


def _project(node_feats, W_in, b_in, W_node0):
    return pl.pallas_call(
        _proj_body,
        grid=(N // BN,),
        in_specs=[
            pl.BlockSpec((BN, NODE_IN), lambda i: (i, 0)),
            pl.BlockSpec((NODE_IN, D), lambda i: (0, 0)),
            pl.BlockSpec((1, D), lambda i: (0, 0)),
            pl.BlockSpec((D, D), lambda i: (0, 0)),
        ],
        out_specs=[
            pl.BlockSpec((BN, D), lambda i: (i, 0)),
            pl.BlockSpec((BN, D), lambda i: (i, 0)),
        ],
        out_shape=[
            jax.ShapeDtypeStruct((N, D), jnp.float32),
            jax.ShapeDtypeStruct((N, D), jnp.float32),
        ],
    )(node_feats, W_in, b_in.reshape(1, D), W_node0)


def _edge_body(ef_ref, we_ref, out_ref):
    out_ref[0] = jnp.dot(ef_ref[...], we_ref[0],
                         preferred_element_type=jnp.float32)


def _edge_transform(edge_feats, W_edge):
    ein = edge_feats.shape[1]
    return pl.pallas_call(
        _edge_body,
        grid=(L, E // BE),
        in_specs=[
            pl.BlockSpec((BE, ein), lambda l, e: (e, 0)),
            pl.BlockSpec((1, ein, D), lambda l, e: (l, 0, 0)),
        ],
        out_specs=pl.BlockSpec((1, BE, D), lambda l, e: (l, e, 0)),
        out_shape=jax.ShapeDtypeStruct((L, E, D), jnp.float32),
    )(edge_feats, W_edge)


def _upd_body(h_ref, p_ref, wt_ref, wb_ref, b_ref, wn_ref, hnew_ref, hw_ref):
    p = p_ref[0] + p_ref[1]
    hn = jnp.maximum(
        jnp.dot(h_ref[...], wt_ref[...], preferred_element_type=jnp.float32)
        + jnp.dot(p, wb_ref[...], preferred_element_type=jnp.float32)
        + b_ref[...], 0.0)
    hnew_ref[...] = hn
    hw_ref[...] = jnp.dot(hn, wn_ref[...], preferred_element_type=jnp.float32)


def _update(h, parts, W_top, W_bot, b, W_node_next):
    return pl.pallas_call(
        _upd_body,
        grid=(N // BN,),
        in_specs=[
            pl.BlockSpec((BN, D), lambda i: (i, 0)),
            pl.BlockSpec((NC, BN, D), lambda i: (0, i, 0)),
            pl.BlockSpec((D, D), lambda i: (0, 0)),
            pl.BlockSpec((D, D), lambda i: (0, 0)),
            pl.BlockSpec((1, D), lambda i: (0, 0)),
            pl.BlockSpec((D, D), lambda i: (0, 0)),
        ],
        out_specs=[
            pl.BlockSpec((BN, D), lambda i: (i, 0)),
            pl.BlockSpec((BN, D), lambda i: (i, 0)),
        ],
        out_shape=[
            jax.ShapeDtypeStruct((N, D), jnp.float32),
            jax.ShapeDtypeStruct((N, D), jnp.float32),
        ],
    )(h, parts, W_top, W_bot, b.reshape(1, D), W_node_next)


def _sc_body(layer, hw_hbm, ew_hbm, src_hbm, dst_hbm, out_hbm, acc,
             src0, dst0, rows0, ew0, src1, dst1, rows1, ew1,
             gsem0, esem0, ssem0, gsem1, esem1, ssem1):
    c = lax.axis_index("c")
    s = lax.axis_index("s")
    bufs = ((src0, dst0, rows0, ew0, gsem0, esem0, ssem0),
            (src1, dst1, rows1, ew1, gsem1, esem1, ssem1))

    zeros16 = jnp.zeros((16,), jnp.float32)

    def zfill(i, _):
        for j in range(D // 16):
            rows0[i, pl.ds(j * 16, 16)] = zeros16
        return 0

    lax.fori_loop(0, CH, zfill, 0)
    for j in range(ROWS_PER_TILE // CH):
        pltpu.sync_copy(rows0, acc.at[pl.ds(s * ROWS_PER_TILE + j * CH, CH)])
    plsc.subcore_barrier()

    base0 = c * EDGES_PER_CORE + s * EDGES_PER_TILE

    def wait_scatter(b):
        _, dst_v, rows_v, _, _, _, ssem = bufs[b]
        pltpu.make_async_copy(rows_v, acc.at[dst_v], ssem).wait()

    def start(i, b):
        src_v, dst_v, rows_v, ew_v, gsem, esem, _ = bufs[b]
        base = base0 + i * CH
        pltpu.sync_copy(src_hbm.at[pl.ds(base, CH)], src_v)
        pltpu.sync_copy(dst_hbm.at[pl.ds(base, CH)], dst_v)
        pltpu.async_copy(hw_hbm.at[src_v], rows_v, gsem)
        pltpu.async_copy(ew_hbm.at[pl.ds(layer * E + base, CH)], ew_v, esem)

    def finish(i, b):
        src_v, dst_v, rows_v, ew_v, gsem, esem, ssem = bufs[b]
        base = base0 + i * CH
        pltpu.make_async_copy(hw_hbm.at[src_v], rows_v, gsem).wait()
        pltpu.make_async_copy(
            ew_hbm.at[pl.ds(layer * E + base, CH)], ew_v, esem).wait()

        @plsc.parallel_loop(0, CH, 1, unroll=4)
        def mul(e):
            for j in range(D // 16):
                sl = pl.ds(j * 16, 16)
                rows_v[e, sl] = rows_v[e, sl] * ew_v[e, sl]

        pltpu.async_copy(rows_v, acc.at[dst_v], ssem, add=True)

    start(0, 0)
    start(1, 1)

    def pair(g, _):
        i0 = 2 * g
        finish(i0, 0)

        @pl.when(i0 + 2 < CHUNKS)
        def _():
            wait_scatter(0)
            start(i0 + 2, 0)

        @pl.when(i0 + 1 < CHUNKS)
        def _():
            finish(i0 + 1, 1)

        @pl.when(i0 + 3 < CHUNKS)
        def _():
            wait_scatter(1)
            start(i0 + 3, 1)

        return 0

    lax.fori_loop(0, (CHUNKS + 1) // 2, pair, 0)
    wait_scatter(0)
    wait_scatter(1)
    plsc.subcore_barrier()

    pltpu.sync_copy(acc.at[pl.ds(s * ROWS_PER_TILE, ROWS_PER_TILE)],
                    out_hbm.at[c, pl.ds(s * ROWS_PER_TILE, ROWS_PER_TILE)])


def _sc_message_pass(layer, hw, ew_flat, src, dst):
    mesh = plsc.VectorSubcoreMesh(core_axis_name="c", subcore_axis_name="s")
    return pl.kernel(
        functools.partial(_sc_body, layer),
        out_type=jax.ShapeDtypeStruct((NC, N_PAD, D), jnp.float32),
        mesh=mesh,
        scratch_types=[
            pltpu.VMEM_SHARED((N_PAD, D), jnp.float32),
            pltpu.VMEM((CH,), jnp.int32),
            pltpu.VMEM((CH,), jnp.int32),
            pltpu.VMEM((CH, D), jnp.float32),
            pltpu.VMEM((CH, D), jnp.float32),
            pltpu.VMEM((CH,), jnp.int32),
            pltpu.VMEM((CH,), jnp.int32),
            pltpu.VMEM((CH, D), jnp.float32),
            pltpu.VMEM((CH, D), jnp.float32),
            pltpu.SemaphoreType.DMA,
            pltpu.SemaphoreType.DMA,
            pltpu.SemaphoreType.DMA,
            pltpu.SemaphoreType.DMA,
            pltpu.SemaphoreType.DMA,
            pltpu.SemaphoreType.DMA,
        ],
    )(hw, ew_flat, src, dst)


def kernel(node_feats, edge_feats, edge_index, W_in, b_in, W_node, W_edge,
           W_new, b_new):
    src = edge_index[0]
    dst = edge_index[1]
    ew_flat = _edge_transform(edge_feats, W_edge).reshape(L * E, D)
    h, hw = _project(node_feats, W_in, b_in, W_node[0])
    for l in range(L):
        parts = _sc_message_pass(l, hw, ew_flat, src, dst)
        wn_next = W_node[(l + 1) % L]
        h, hw = _update(h, parts, W_new[l][:D], W_new[l][D:], b_new[l], wn_next)
    return h
